# baseline (speedup 1.0000x reference)
import functools

import jax
import jax.numpy as jnp
from jax import lax
from jax.experimental import pallas as pl
from jax.experimental.pallas import tpu as pltpu
from jax.experimental.pallas import tpu_sc as plsc

N_NODES = 10000
N_EDGES = 320000
D_FEAT = 128
DIM_H = 128
N_CLASSES = 2

NC = 2
NS = 16
CH = 96
N_ROWS = 10096
ROWS_PER_TILE = N_ROWS // NS
NCHUNK = 105
E_PAD = NC * NS * NCHUNK * CH
G = 21
PAD_DST = N_NODES

_mesh = plsc.VectorSubcoreMesh(core_axis_name="c", subcore_axis_name="s")


def _sc_degree(dst2d, ones_block, zeros_rows):
    @functools.partial(
        pl.kernel,
        out_type=jax.ShapeDtypeStruct((NC * N_ROWS, 16), jnp.float32),
        mesh=_mesh,
        scratch_types=[
            pltpu.VMEM((NCHUNK, CH), jnp.int32),
            pltpu.VMEM((CH, 16), jnp.float32),
            pltpu.VMEM_SHARED((N_ROWS, 16), jnp.float32),
        ],
        compiler_params=pltpu.CompilerParams(use_tc_tiling_on_sc=False),
    )
    def k(dst_hbm, ones_hbm, zeros_hbm, out_hbm, di, ones_v, acc):
        core = lax.axis_index("c")
        tid = lax.axis_index("s")
        cbase = (core * NS + tid) * NCHUNK
        pltpu.sync_copy(zeros_hbm,
                        acc.at[pl.ds(tid * ROWS_PER_TILE, ROWS_PER_TILE)])
        pltpu.sync_copy(ones_hbm, ones_v)
        pltpu.sync_copy(dst_hbm.at[pl.ds(cbase, NCHUNK)], di)
        plsc.subcore_barrier()

        @pl.loop(0, NCHUNK)
        def _(i):
            pltpu.sync_copy(ones_v, acc.at[di.at[i]], add=True)

        plsc.subcore_barrier()
        pltpu.sync_copy(
            acc.at[pl.ds(tid * ROWS_PER_TILE, ROWS_PER_TILE)],
            out_hbm.at[pl.ds(core * N_ROWS + tid * ROWS_PER_TILE, ROWS_PER_TILE)])

    return k(dst2d, ones_block, zeros_rows)


def _sc_aggregate(s_table, src, dst, zeros_rows, d):
    dt = s_table.dtype

    @functools.partial(
        pl.kernel,
        out_type=jax.ShapeDtypeStruct((NC * N_ROWS, d), dt),
        mesh=_mesh,
        name=f"sc_aggregate_d{d}",
        scratch_types=[
            pltpu.VMEM((G, CH), jnp.int32),
            pltpu.VMEM((G, CH), jnp.int32),
            pltpu.VMEM((CH, d), dt),
            pltpu.VMEM((CH, d), dt),
            pltpu.VMEM((CH, d), dt),
            pltpu.VMEM_SHARED((N_ROWS, d), dt),
            pltpu.VMEM_SHARED((N_NODES, d), dt),
            pltpu.SemaphoreType.DMA,
            pltpu.SemaphoreType.DMA,
            pltpu.SemaphoreType.DMA,
        ],
        compiler_params=pltpu.CompilerParams(use_tc_tiling_on_sc=False),
    )
    def k(s_hbm, src_hbm, dst_hbm, zeros_hbm, out_hbm,
          si, di, rows0, rows1, rows2, acc, stab, gsem0, gsem1, gsem2):
        core = lax.axis_index("c")
        tid = lax.axis_index("s")
        cbase = (core * NS + tid) * NCHUNK
        with jax.named_scope("zero_fill"):
            pltpu.sync_copy(zeros_hbm,
                            acc.at[pl.ds(tid * ROWS_PER_TILE, ROWS_PER_TILE)])
            pltpu.sync_copy(
                s_hbm.at[pl.ds(tid * (N_NODES // NS), N_NODES // NS)],
                stab.at[pl.ds(tid * (N_NODES // NS), N_NODES // NS)])
            plsc.subcore_barrier()

        @pl.loop(0, NCHUNK // G)
        def _(g):
            pltpu.sync_copy(src_hbm.at[pl.ds(cbase + g * G, G)], si)
            pltpu.sync_copy(dst_hbm.at[pl.ds(cbase + g * G, G)], di)
            pltpu.async_copy(stab.at[si.at[0]], rows0, gsem0)
            pltpu.async_copy(stab.at[si.at[1]], rows1, gsem1)

            @pl.loop(0, G // 3)
            def _(j):
                i = 3 * j
                pltpu.async_copy(stab.at[si.at[i + 2]], rows2, gsem2)
                pltpu.make_async_copy(stab.at[si.at[i]], rows0, gsem0).wait()
                pltpu.sync_copy(rows0, acc.at[di.at[i]], add=True)

                @pl.when(i + 3 < G)
                def _():
                    pltpu.async_copy(stab.at[si.at[i + 3]], rows0, gsem0)

                pltpu.make_async_copy(stab.at[si.at[i + 1]], rows1, gsem1).wait()
                pltpu.sync_copy(rows1, acc.at[di.at[i + 1]], add=True)

                @pl.when(i + 4 < G)
                def _():
                    pltpu.async_copy(stab.at[si.at[i + 4]], rows1, gsem1)

                pltpu.make_async_copy(stab.at[si.at[i + 2]], rows2, gsem2).wait()
                pltpu.sync_copy(rows2, acc.at[di.at[i + 2]], add=True)

        with jax.named_scope("copy_out"):
            plsc.subcore_barrier()
            pltpu.sync_copy(
                acc.at[pl.ds(tid * ROWS_PER_TILE, ROWS_PER_TILE)],
                out_hbm.at[pl.ds(core * N_ROWS + tid * ROWS_PER_TILE, ROWS_PER_TILE)])

    return k(s_table, src, dst, zeros_rows)


ROW_BLK = 1000


def _tc_matmul(x, w):
    n, kdim = x.shape
    m = w.shape[1]

    def body(x_ref, w_ref, o_ref):
        o_ref[...] = jnp.dot(x_ref[...], w_ref[...],
                             preferred_element_type=jnp.float32,
                             precision=lax.Precision.HIGHEST)

    return pl.pallas_call(
        body,
        grid=(n // ROW_BLK,),
        in_specs=[
            pl.BlockSpec((ROW_BLK, kdim), lambda i: (i, 0)),
            pl.BlockSpec((kdim, m), lambda i: (0, 0)),
        ],
        out_specs=pl.BlockSpec((ROW_BLK, m), lambda i: (i, 0)),
        out_shape=jax.ShapeDtypeStruct((n, m), jnp.float32),
    )(x, w)


def _dinv_block(degp_ref):
    deg = degp_ref[0, :, 0:1] + degp_ref[1, :, 0:1] + 1.0
    return lax.rsqrt(deg)


def _tc_scale(xw, deg_parts):
    def body(xw_ref, degp_ref, o_ref):
        o_ref[...] = (xw_ref[...] * _dinv_block(degp_ref)).astype(jnp.bfloat16)

    return pl.pallas_call(
        body,
        grid=(N_NODES // ROW_BLK,),
        in_specs=[
            pl.BlockSpec((ROW_BLK, DIM_H), lambda i: (i, 0)),
            pl.BlockSpec((NC, ROW_BLK, 16), lambda i: (0, i, 0)),
        ],
        out_specs=pl.BlockSpec((ROW_BLK, DIM_H), lambda i: (i, 0)),
        out_shape=jax.ShapeDtypeStruct((N_NODES, DIM_H), jnp.bfloat16),
    )(xw, deg_parts)


def _tc_mid(y_parts, xw, deg_parts, b1, w2pad):
    def body(yp_ref, xw_ref, degp_ref, b1_ref, w2_ref, hw_ref, s2_ref):
        dinv = _dinv_block(degp_ref)
        y = yp_ref[0].astype(jnp.float32) + yp_ref[1].astype(jnp.float32)
        h = jnp.maximum(dinv * y + (dinv * dinv) * xw_ref[...] + b1_ref[...], 0.0)
        hw = jnp.dot(h, w2_ref[...], preferred_element_type=jnp.float32,
                     precision=lax.Precision.HIGHEST)
        hw_ref[...] = hw
        s2_ref[...] = hw * dinv

    return pl.pallas_call(
        body,
        grid=(N_NODES // ROW_BLK,),
        in_specs=[
            pl.BlockSpec((NC, ROW_BLK, DIM_H), lambda i: (0, i, 0)),
            pl.BlockSpec((ROW_BLK, DIM_H), lambda i: (i, 0)),
            pl.BlockSpec((NC, ROW_BLK, 16), lambda i: (0, i, 0)),
            pl.BlockSpec((1, DIM_H), lambda i: (0, 0)),
            pl.BlockSpec((DIM_H, 16), lambda i: (0, 0)),
        ],
        out_specs=[
            pl.BlockSpec((ROW_BLK, 16), lambda i: (i, 0)),
            pl.BlockSpec((ROW_BLK, 16), lambda i: (i, 0)),
        ],
        out_shape=[
            jax.ShapeDtypeStruct((N_NODES, 16), jnp.float32),
            jax.ShapeDtypeStruct((N_NODES, 16), jnp.float32),
        ],
    )(y_parts, xw, deg_parts, b1, w2pad)


def _tc_final(y2_parts, hwpad, deg_parts, b2pad):
    def body(yp_ref, hw_ref, degp_ref, b2_ref, o_ref):
        dinv = _dinv_block(degp_ref)
        y = yp_ref[0] + yp_ref[1]
        z = dinv * y + (dinv * dinv) * hw_ref[...] + b2_ref[...]
        z0 = z[:, 0:1]
        z1 = z[:, 1:2]
        m = jnp.maximum(z0, z1)
        lse = m + jnp.log(jnp.exp(z0 - m) + jnp.exp(z1 - m))
        o_ref[...] = (z - lse)[:, :N_CLASSES]

    return pl.pallas_call(
        body,
        grid=(N_NODES // ROW_BLK,),
        in_specs=[
            pl.BlockSpec((NC, ROW_BLK, 16), lambda i: (0, i, 0)),
            pl.BlockSpec((ROW_BLK, 16), lambda i: (i, 0)),
            pl.BlockSpec((NC, ROW_BLK, 16), lambda i: (0, i, 0)),
            pl.BlockSpec((1, 16), lambda i: (0, 0)),
        ],
        out_specs=pl.BlockSpec((ROW_BLK, N_CLASSES), lambda i: (i, 0)),
        out_shape=jax.ShapeDtypeStruct((N_NODES, N_CLASSES), jnp.float32),
    )(y2_parts, hwpad, deg_parts, b2pad)


def kernel(x, edge_index, W1, b1, W2, b2):
    npad = E_PAD - N_EDGES
    pad_src = jnp.arange(npad, dtype=jnp.int32) % N_NODES
    pad_dst = PAD_DST + (jnp.arange(npad, dtype=jnp.int32) % (N_ROWS - N_NODES))
    src = jnp.concatenate(
        [edge_index[0].astype(jnp.int32), pad_src]).reshape(E_PAD // CH, CH)
    dst = jnp.concatenate(
        [edge_index[1].astype(jnp.int32), pad_dst]).reshape(E_PAD // CH, CH)

    ones_block = jnp.ones((CH, 16), jnp.float32)
    zeros_128 = jnp.zeros((ROWS_PER_TILE, DIM_H), jnp.bfloat16)
    zeros_16 = jnp.zeros((ROWS_PER_TILE, 16), jnp.float32)
    w2pad = jnp.pad(W2, ((0, 0), (0, 16 - N_CLASSES)))
    b1r = b1.reshape(1, DIM_H)
    b2pad = jnp.pad(b2, (0, 16 - N_CLASSES)).reshape(1, 16)

    deg_flat = _sc_degree(dst, ones_block, zeros_16)
    deg_parts = deg_flat.reshape(NC, N_ROWS, 16)[:, :N_NODES, :]

    xw = _tc_matmul(x, W1)
    s1 = _tc_scale(xw, deg_parts)

    y1 = _sc_aggregate(s1, src, dst, zeros_128, DIM_H)
    y1_parts = y1.reshape(NC, N_ROWS, DIM_H)[:, :N_NODES, :]

    hwpad, s2pad = _tc_mid(y1_parts, xw, deg_parts, b1r, w2pad)

    y2 = _sc_aggregate(s2pad, src, dst, zeros_16, 16)
    y2_parts = y2.reshape(NC, N_ROWS, 16)[:, :N_NODES, :]

    return _tc_final(y2_parts, hwpad, deg_parts, b2pad)


# speedup vs baseline: 1.0230x; 1.0230x over previous
import functools

import jax
import jax.numpy as jnp
from jax import lax
from jax.experimental import pallas as pl
from jax.experimental.pallas import tpu as pltpu
from jax.experimental.pallas import tpu_sc as plsc

N_NODES = 10000
N_EDGES = 320000
D_FEAT = 128
DIM_H = 128
N_CLASSES = 2

NC = 2
NS = 16
CH = 96
N_ROWS = 10096
ROWS_PER_TILE = N_ROWS // NS
NCHUNK = 105
E_PAD = NC * NS * NCHUNK * CH
G = 21
PAD_DST = N_NODES

_mesh = plsc.VectorSubcoreMesh(core_axis_name="c", subcore_axis_name="s")


def _sc_degree(dst2d, ones_block, zeros_rows):
    @functools.partial(
        pl.kernel,
        out_type=jax.ShapeDtypeStruct((NC * N_ROWS, 16), jnp.float32),
        mesh=_mesh,
        scratch_types=[
            pltpu.VMEM((NCHUNK, CH), jnp.int32),
            pltpu.VMEM((CH, 16), jnp.float32),
            pltpu.VMEM_SHARED((N_ROWS, 16), jnp.float32),
        ],
        compiler_params=pltpu.CompilerParams(use_tc_tiling_on_sc=False),
    )
    def k(dst_hbm, ones_hbm, zeros_hbm, out_hbm, di, ones_v, acc):
        core = lax.axis_index("c")
        tid = lax.axis_index("s")
        cbase = (core * NS + tid) * NCHUNK
        pltpu.sync_copy(zeros_hbm,
                        acc.at[pl.ds(tid * ROWS_PER_TILE, ROWS_PER_TILE)])
        pltpu.sync_copy(ones_hbm, ones_v)
        pltpu.sync_copy(dst_hbm.at[pl.ds(cbase, NCHUNK)], di)
        plsc.subcore_barrier()

        @pl.loop(0, NCHUNK)
        def _(i):
            pltpu.sync_copy(ones_v, acc.at[di.at[i]], add=True)

        plsc.subcore_barrier()
        pltpu.sync_copy(
            acc.at[pl.ds(tid * ROWS_PER_TILE, ROWS_PER_TILE)],
            out_hbm.at[pl.ds(core * N_ROWS + tid * ROWS_PER_TILE, ROWS_PER_TILE)])

    return k(dst2d, ones_block, zeros_rows)


def _sc_aggregate(s_table, src, dst, zeros_rows, d):
    dt = s_table.dtype

    @functools.partial(
        pl.kernel,
        out_type=jax.ShapeDtypeStruct((NC * N_ROWS, d), dt),
        mesh=_mesh,
        name=f"sc_aggregate_d{d}",
        scratch_types=[
            pltpu.VMEM((G, CH), jnp.int32),
            pltpu.VMEM((G, CH), jnp.int32),
            pltpu.VMEM((CH, d), dt),
            pltpu.VMEM((CH, d), dt),
            pltpu.VMEM((CH, d), dt),
            pltpu.VMEM_SHARED((N_ROWS, d), dt),
            pltpu.SemaphoreType.DMA,
            pltpu.SemaphoreType.DMA,
            pltpu.SemaphoreType.DMA,
        ],
        compiler_params=pltpu.CompilerParams(use_tc_tiling_on_sc=False),
    )
    def k(s_hbm, src_hbm, dst_hbm, zeros_hbm, out_hbm,
          si, di, rows0, rows1, rows2, acc, gsem0, gsem1, gsem2):
        core = lax.axis_index("c")
        tid = lax.axis_index("s")
        cbase = (core * NS + tid) * NCHUNK
        with jax.named_scope("zero_fill"):
            pltpu.sync_copy(zeros_hbm,
                            acc.at[pl.ds(tid * ROWS_PER_TILE, ROWS_PER_TILE)])
            plsc.subcore_barrier()

        @pl.loop(0, NCHUNK // G)
        def _(g):
            pltpu.sync_copy(src_hbm.at[pl.ds(cbase + g * G, G)], si)
            pltpu.sync_copy(dst_hbm.at[pl.ds(cbase + g * G, G)], di)
            pltpu.async_copy(s_hbm.at[si.at[0]], rows0, gsem0)
            pltpu.async_copy(s_hbm.at[si.at[1]], rows1, gsem1)

            @pl.loop(0, G // 3)
            def _(j):
                i = 3 * j
                pltpu.async_copy(s_hbm.at[si.at[i + 2]], rows2, gsem2)
                pltpu.make_async_copy(s_hbm.at[si.at[i]], rows0, gsem0).wait()
                pltpu.sync_copy(rows0, acc.at[di.at[i]], add=True)

                @pl.when(i + 3 < G)
                def _():
                    pltpu.async_copy(s_hbm.at[si.at[i + 3]], rows0, gsem0)

                pltpu.make_async_copy(s_hbm.at[si.at[i + 1]], rows1, gsem1).wait()
                pltpu.sync_copy(rows1, acc.at[di.at[i + 1]], add=True)

                @pl.when(i + 4 < G)
                def _():
                    pltpu.async_copy(s_hbm.at[si.at[i + 4]], rows1, gsem1)

                pltpu.make_async_copy(s_hbm.at[si.at[i + 2]], rows2, gsem2).wait()
                pltpu.sync_copy(rows2, acc.at[di.at[i + 2]], add=True)

        with jax.named_scope("copy_out"):
            plsc.subcore_barrier()
            pltpu.sync_copy(
                acc.at[pl.ds(tid * ROWS_PER_TILE, ROWS_PER_TILE)],
                out_hbm.at[pl.ds(core * N_ROWS + tid * ROWS_PER_TILE, ROWS_PER_TILE)])

    return k(s_table, src, dst, zeros_rows)


ROW_BLK = 1000


def _tc_matmul(x, w):
    n, kdim = x.shape
    m = w.shape[1]

    def body(x_ref, w_ref, o_ref):
        o_ref[...] = jnp.dot(x_ref[...], w_ref[...],
                             preferred_element_type=jnp.float32,
                             precision=lax.Precision.HIGHEST)

    return pl.pallas_call(
        body,
        grid=(n // ROW_BLK,),
        in_specs=[
            pl.BlockSpec((ROW_BLK, kdim), lambda i: (i, 0)),
            pl.BlockSpec((kdim, m), lambda i: (0, 0)),
        ],
        out_specs=pl.BlockSpec((ROW_BLK, m), lambda i: (i, 0)),
        out_shape=jax.ShapeDtypeStruct((n, m), jnp.float32),
    )(x, w)


def _dinv_block(degp_ref):
    deg = degp_ref[0, :, 0:1] + degp_ref[1, :, 0:1] + 1.0
    return lax.rsqrt(deg)


def _tc_scale(xw, deg_parts):
    def body(xw_ref, degp_ref, o_ref):
        o_ref[...] = (xw_ref[...] * _dinv_block(degp_ref)).astype(jnp.bfloat16)

    return pl.pallas_call(
        body,
        grid=(N_NODES // ROW_BLK,),
        in_specs=[
            pl.BlockSpec((ROW_BLK, DIM_H), lambda i: (i, 0)),
            pl.BlockSpec((NC, ROW_BLK, 16), lambda i: (0, i, 0)),
        ],
        out_specs=pl.BlockSpec((ROW_BLK, DIM_H), lambda i: (i, 0)),
        out_shape=jax.ShapeDtypeStruct((N_NODES, DIM_H), jnp.bfloat16),
    )(xw, deg_parts)


def _tc_mid(y_parts, xw, deg_parts, b1, w2pad):
    def body(yp_ref, xw_ref, degp_ref, b1_ref, w2_ref, hw_ref, s2_ref):
        dinv = _dinv_block(degp_ref)
        y = yp_ref[0].astype(jnp.float32) + yp_ref[1].astype(jnp.float32)
        h = jnp.maximum(dinv * y + (dinv * dinv) * xw_ref[...] + b1_ref[...], 0.0)
        hw = jnp.dot(h, w2_ref[...], preferred_element_type=jnp.float32,
                     precision=lax.Precision.HIGHEST)
        hw_ref[...] = hw
        s2_ref[...] = hw * dinv

    return pl.pallas_call(
        body,
        grid=(N_NODES // ROW_BLK,),
        in_specs=[
            pl.BlockSpec((NC, ROW_BLK, DIM_H), lambda i: (0, i, 0)),
            pl.BlockSpec((ROW_BLK, DIM_H), lambda i: (i, 0)),
            pl.BlockSpec((NC, ROW_BLK, 16), lambda i: (0, i, 0)),
            pl.BlockSpec((1, DIM_H), lambda i: (0, 0)),
            pl.BlockSpec((DIM_H, 16), lambda i: (0, 0)),
        ],
        out_specs=[
            pl.BlockSpec((ROW_BLK, 16), lambda i: (i, 0)),
            pl.BlockSpec((ROW_BLK, 16), lambda i: (i, 0)),
        ],
        out_shape=[
            jax.ShapeDtypeStruct((N_NODES, 16), jnp.float32),
            jax.ShapeDtypeStruct((N_NODES, 16), jnp.float32),
        ],
    )(y_parts, xw, deg_parts, b1, w2pad)


def _tc_final(y2_parts, hwpad, deg_parts, b2pad):
    def body(yp_ref, hw_ref, degp_ref, b2_ref, o_ref):
        dinv = _dinv_block(degp_ref)
        y = yp_ref[0] + yp_ref[1]
        z = dinv * y + (dinv * dinv) * hw_ref[...] + b2_ref[...]
        z0 = z[:, 0:1]
        z1 = z[:, 1:2]
        m = jnp.maximum(z0, z1)
        lse = m + jnp.log(jnp.exp(z0 - m) + jnp.exp(z1 - m))
        o_ref[...] = (z - lse)[:, :N_CLASSES]

    return pl.pallas_call(
        body,
        grid=(N_NODES // ROW_BLK,),
        in_specs=[
            pl.BlockSpec((NC, ROW_BLK, 16), lambda i: (0, i, 0)),
            pl.BlockSpec((ROW_BLK, 16), lambda i: (i, 0)),
            pl.BlockSpec((NC, ROW_BLK, 16), lambda i: (0, i, 0)),
            pl.BlockSpec((1, 16), lambda i: (0, 0)),
        ],
        out_specs=pl.BlockSpec((ROW_BLK, N_CLASSES), lambda i: (i, 0)),
        out_shape=jax.ShapeDtypeStruct((N_NODES, N_CLASSES), jnp.float32),
    )(y2_parts, hwpad, deg_parts, b2pad)


def kernel(x, edge_index, W1, b1, W2, b2):
    npad = E_PAD - N_EDGES
    pad_src = jnp.arange(npad, dtype=jnp.int32) % N_NODES
    pad_dst = PAD_DST + (jnp.arange(npad, dtype=jnp.int32) % (N_ROWS - N_NODES))
    src = jnp.concatenate(
        [edge_index[0].astype(jnp.int32), pad_src]).reshape(E_PAD // CH, CH)
    dst = jnp.concatenate(
        [edge_index[1].astype(jnp.int32), pad_dst]).reshape(E_PAD // CH, CH)

    ones_block = jnp.ones((CH, 16), jnp.float32)
    zeros_128 = jnp.zeros((ROWS_PER_TILE, DIM_H), jnp.bfloat16)
    zeros_16 = jnp.zeros((ROWS_PER_TILE, 16), jnp.float32)
    w2pad = jnp.pad(W2, ((0, 0), (0, 16 - N_CLASSES)))
    b1r = b1.reshape(1, DIM_H)
    b2pad = jnp.pad(b2, (0, 16 - N_CLASSES)).reshape(1, 16)

    deg_flat = _sc_degree(dst, ones_block, zeros_16)
    deg_parts = deg_flat.reshape(NC, N_ROWS, 16)[:, :N_NODES, :]

    xw = _tc_matmul(x, W1)
    s1 = _tc_scale(xw, deg_parts)

    y1 = _sc_aggregate(s1, src, dst, zeros_128, DIM_H)
    y1_parts = y1.reshape(NC, N_ROWS, DIM_H)[:, :N_NODES, :]

    hwpad, s2pad = _tc_mid(y1_parts, xw, deg_parts, b1r, w2pad)

    y2 = _sc_aggregate(s2pad, src, dst, zeros_16, 16)
    y2_parts = y2.reshape(NC, N_ROWS, 16)[:, :N_NODES, :]

    return _tc_final(y2_parts, hwpad, deg_parts, b2pad)
